# baseline (speedup 1.0000x reference)
import functools

import jax
import jax.numpy as jnp
from jax import lax
from jax.experimental import pallas as pl
from jax.experimental.pallas import tpu as pltpu
from jax.experimental.pallas import tpu_sc as plsc

N = 10000
D = 128
K = 16
ROWS_BLK = 200

S = 5
NS_R = N // S

_NC = 2
_NS = 16
_NW = _NC * _NS
NPAD = 2048
RPW = NPAD // _NW
RC = 8


def _knn_body(x_ref, xt_ref, idx_ref, d_ref, *, row0):
    i = pl.program_id(0)
    xb = x_ref[...]
    xt = xt_ref[...]
    sqj = jnp.sum(xt * xt, axis=0, keepdims=True)
    sqi = jnp.sum(xb * xb, axis=1, keepdims=True)
    prod = jnp.dot(
        xb.astype(jnp.bfloat16),
        xt.astype(jnp.bfloat16),
        preferred_element_type=jnp.float32,
    )
    d = (sqi - 2.0 * prod) + sqj
    colf = lax.broadcasted_iota(jnp.int32, (ROWS_BLK, N), 1).astype(jnp.float32)
    row_g = row0 + i * ROWS_BLK + lax.broadcasted_iota(
        jnp.int32, (ROWS_BLK, N), 0
    )
    rowf = row_g.astype(jnp.float32)
    d_ref[...] = jnp.where(colf == rowf, jnp.inf, d)

    cols = []
    for t in range(K):
        dcur = d_ref[...]
        m = jnp.min(dcur, axis=1, keepdims=True)
        masked_col = jnp.where(dcur == m, colf, jnp.float32(N))
        chosen = jnp.min(masked_col, axis=1, keepdims=True)
        cols.append(chosen.astype(jnp.int32))
        if t < K - 1:
            d_ref[...] = jnp.where(colf == chosen, jnp.inf, dcur)
    idx_ref[...] = jnp.concatenate(cols, axis=1)


def _knn_tc(x, xt, row0):
    return pl.pallas_call(
        functools.partial(_knn_body, row0=row0),
        grid=(NS_R // ROWS_BLK,),
        in_specs=[
            pl.BlockSpec((ROWS_BLK, D), lambda i: (i, 0)),
            pl.BlockSpec((D, N), lambda i: (0, 0)),
        ],
        out_specs=pl.BlockSpec((ROWS_BLK, K), lambda i: (i, 0)),
        out_shape=jax.ShapeDtypeStruct((NS_R, K), jnp.int32),
        scratch_shapes=[pltpu.VMEM((ROWS_BLK, N), jnp.float32)],
    )(x, xt)


def _gather_body(x_hbm, idx_hbm, out_hbm, idx_v, rows_v, sem0, sem1):
    wid = lax.axis_index("s") * _NC + lax.axis_index("c")
    base = wid * RPW
    nch = RPW // RC
    sems = (sem0, sem1)
    handles = [None, None]

    pltpu.sync_copy(idx_hbm.at[pl.ds(base * K, RC * K)], idx_v.at[0])
    handles[0] = pltpu.async_copy(x_hbm.at[idx_v.at[0]], rows_v.at[0], sems[0])
    for ci in range(1, nch + 1):
        b = ci % 2
        pb = (ci - 1) % 2
        if ci < nch:
            r = base + ci * RC
            pltpu.sync_copy(idx_hbm.at[pl.ds(r * K, RC * K)], idx_v.at[b])
            handles[b] = pltpu.async_copy(
                x_hbm.at[idx_v.at[b]], rows_v.at[b], sems[b]
            )
        handles[pb].wait()
        rp = base + (ci - 1) * RC
        pltpu.sync_copy(rows_v.at[pb], out_hbm.at[pl.ds(rp * K, RC * K)])


def _gather_sc(x, idx_flat):
    mesh = plsc.VectorSubcoreMesh(core_axis_name="c", subcore_axis_name="s")
    fn = functools.partial(
        pl.kernel,
        out_type=jax.ShapeDtypeStruct((NPAD * K, D), jnp.float32),
        mesh=mesh,
        scratch_types=[
            pltpu.VMEM((2, RC * K), jnp.int32),
            pltpu.VMEM((2, RC * K, D), jnp.float32),
            pltpu.SemaphoreType.DMA,
            pltpu.SemaphoreType.DMA,
        ],
    )(_gather_body)
    return fn(x, idx_flat)


def _mlp_body(x_ref, xj_ref, w_ref, b_ref, out_ref):
    xb = x_ref[...]
    e1 = xb.astype(jnp.bfloat16)
    w = w_ref[...]
    b = b_ref[...]
    m = None
    for j in range(K):
        xj = xj_ref[:, j, :]
        e2 = (xj - xb).astype(jnp.bfloat16)
        ej = jnp.concatenate([e1, e2], axis=1)
        h = jnp.dot(ej, w, preferred_element_type=jnp.float32) + b
        m = h if m is None else jnp.maximum(m, h)
    out_ref[...] = jnp.where(m >= 0.0, m, 0.2 * m)


def _mlp_tc(x, xj, wbf, b):
    return pl.pallas_call(
        _mlp_body,
        grid=(NS_R // ROWS_BLK,),
        in_specs=[
            pl.BlockSpec((ROWS_BLK, D), lambda i: (i, 0)),
            pl.BlockSpec((ROWS_BLK, K, D), lambda i: (i, 0, 0)),
            pl.BlockSpec((2 * D, D), lambda i: (0, 0)),
            pl.BlockSpec((1, D), lambda i: (0, 0)),
        ],
        out_specs=pl.BlockSpec((ROWS_BLK, D), lambda i: (i, 0)),
        out_shape=jax.ShapeDtypeStruct((NS_R, D), jnp.float32),
    )(x, xj, wbf, b)


def kernel(x, W1, b1, W2, b2, W3, b3):
    h = x
    for W, b in ((W1, b1), (W2, b2), (W3, b3)):
        ht = h.T
        wbf = W.astype(jnp.bfloat16)
        idxs = [
            _knn_tc(h[s * NS_R : (s + 1) * NS_R], ht, s * NS_R)
            for s in range(S)
        ]
        parts = []
        for s in range(S):
            idx_flat = jnp.pad(idxs[s], ((0, NPAD - NS_R), (0, 0))).reshape(-1)
            xj_flat = _gather_sc(h, idx_flat)
            xj = xj_flat[: NS_R * K].reshape(NS_R, K, D)
            parts.append(
                _mlp_tc(h[s * NS_R : (s + 1) * NS_R], xj, wbf, b[None])
            )
        h = jnp.concatenate(parts, axis=0)
    return h


# speedup vs baseline: 1.0304x; 1.0304x over previous
import functools

import jax
import jax.numpy as jnp
from jax import lax
from jax.experimental import pallas as pl
from jax.experimental.pallas import tpu as pltpu
from jax.experimental.pallas import tpu_sc as plsc

N = 10000
D = 128
K = 16
ROWS_BLK = 200

S = 2
NS_R = N // S

_NC = 2
_NS = 16
_NW = _NC * _NS
NPAD = 5120
RPW = NPAD // _NW
RC = 8


def _knn_body(x_ref, xt_ref, idx_ref, d_ref, *, row0):
    i = pl.program_id(0)
    xb = x_ref[...]
    xt = xt_ref[...]
    sqj = jnp.sum(xt * xt, axis=0, keepdims=True)
    sqi = jnp.sum(xb * xb, axis=1, keepdims=True)
    prod = jnp.dot(
        xb.astype(jnp.bfloat16),
        xt.astype(jnp.bfloat16),
        preferred_element_type=jnp.float32,
    )
    d = (sqi - 2.0 * prod) + sqj
    colf = lax.broadcasted_iota(jnp.int32, (ROWS_BLK, N), 1).astype(jnp.float32)
    row_g = row0 + i * ROWS_BLK + lax.broadcasted_iota(
        jnp.int32, (ROWS_BLK, N), 0
    )
    rowf = row_g.astype(jnp.float32)
    d_ref[...] = jnp.where(colf == rowf, jnp.inf, d)

    cols = []
    for t in range(K):
        dcur = d_ref[...]
        m = jnp.min(dcur, axis=1, keepdims=True)
        masked_col = jnp.where(dcur == m, colf, jnp.float32(N))
        chosen = jnp.min(masked_col, axis=1, keepdims=True)
        cols.append(chosen.astype(jnp.int32))
        if t < K - 1:
            d_ref[...] = jnp.where(colf == chosen, jnp.inf, dcur)
    idx_ref[...] = jnp.concatenate(cols, axis=1)


def _knn_tc(x, xt, row0):
    return pl.pallas_call(
        functools.partial(_knn_body, row0=row0),
        grid=(NS_R // ROWS_BLK,),
        in_specs=[
            pl.BlockSpec((ROWS_BLK, D), lambda i: (i, 0)),
            pl.BlockSpec((D, N), lambda i: (0, 0)),
        ],
        out_specs=pl.BlockSpec((ROWS_BLK, K), lambda i: (i, 0)),
        out_shape=jax.ShapeDtypeStruct((NS_R, K), jnp.int32),
        scratch_shapes=[pltpu.VMEM((ROWS_BLK, N), jnp.float32)],
    )(x, xt)


def _gather_body(x_hbm, idx_hbm, out_hbm, idx_v, rows_v, sem0, sem1):
    wid = lax.axis_index("s") * _NC + lax.axis_index("c")
    base = wid * RPW
    nch = RPW // RC
    sems = (sem0, sem1)
    handles = [None, None]

    pltpu.sync_copy(idx_hbm.at[pl.ds(base * K, RC * K)], idx_v.at[0])
    handles[0] = pltpu.async_copy(x_hbm.at[idx_v.at[0]], rows_v.at[0], sems[0])
    for ci in range(1, nch + 1):
        b = ci % 2
        pb = (ci - 1) % 2
        if ci < nch:
            r = base + ci * RC
            pltpu.sync_copy(idx_hbm.at[pl.ds(r * K, RC * K)], idx_v.at[b])
            handles[b] = pltpu.async_copy(
                x_hbm.at[idx_v.at[b]], rows_v.at[b], sems[b]
            )
        handles[pb].wait()
        rp = base + (ci - 1) * RC
        pltpu.sync_copy(rows_v.at[pb], out_hbm.at[pl.ds(rp * K, RC * K)])


def _gather_sc(x, idx_flat):
    mesh = plsc.VectorSubcoreMesh(core_axis_name="c", subcore_axis_name="s")
    fn = functools.partial(
        pl.kernel,
        out_type=jax.ShapeDtypeStruct((NPAD * K, D), jnp.float32),
        mesh=mesh,
        scratch_types=[
            pltpu.VMEM((2, RC * K), jnp.int32),
            pltpu.VMEM((2, RC * K, D), jnp.float32),
            pltpu.SemaphoreType.DMA,
            pltpu.SemaphoreType.DMA,
        ],
    )(_gather_body)
    return fn(x, idx_flat)


def _mlp_body(x_ref, xj_ref, w_ref, b_ref, out_ref):
    xb = x_ref[...]
    e1 = xb.astype(jnp.bfloat16)
    w = w_ref[...]
    b = b_ref[...]
    m = None
    for j in range(K):
        xj = xj_ref[:, j, :]
        e2 = (xj - xb).astype(jnp.bfloat16)
        ej = jnp.concatenate([e1, e2], axis=1)
        h = jnp.dot(ej, w, preferred_element_type=jnp.float32) + b
        m = h if m is None else jnp.maximum(m, h)
    out_ref[...] = jnp.where(m >= 0.0, m, 0.2 * m)


def _mlp_tc(x, xj, wbf, b):
    return pl.pallas_call(
        _mlp_body,
        grid=(NS_R // ROWS_BLK,),
        in_specs=[
            pl.BlockSpec((ROWS_BLK, D), lambda i: (i, 0)),
            pl.BlockSpec((ROWS_BLK, K, D), lambda i: (i, 0, 0)),
            pl.BlockSpec((2 * D, D), lambda i: (0, 0)),
            pl.BlockSpec((1, D), lambda i: (0, 0)),
        ],
        out_specs=pl.BlockSpec((ROWS_BLK, D), lambda i: (i, 0)),
        out_shape=jax.ShapeDtypeStruct((NS_R, D), jnp.float32),
    )(x, xj, wbf, b)


def kernel(x, W1, b1, W2, b2, W3, b3):
    h = x
    for W, b in ((W1, b1), (W2, b2), (W3, b3)):
        ht = h.T
        wbf = W.astype(jnp.bfloat16)
        idxs = [
            _knn_tc(h[s * NS_R : (s + 1) * NS_R], ht, s * NS_R)
            for s in range(S)
        ]
        parts = []
        for s in range(S):
            idx_flat = jnp.pad(idxs[s], ((0, NPAD - NS_R), (0, 0))).reshape(-1)
            xj_flat = _gather_sc(h, idx_flat)
            xj = xj_flat[: NS_R * K].reshape(NS_R, K, D)
            parts.append(
                _mlp_tc(h[s * NS_R : (s + 1) * NS_R], xj, wbf, b[None])
            )
        h = jnp.concatenate(parts, axis=0)
    return h
